# baseline (speedup 1.0000x reference)
import functools

import jax
import jax.numpy as jnp
from jax import lax
from jax.experimental import pallas as pl
from jax.experimental.pallas import tpu as pltpu
from jax.experimental.pallas import tpu_sc as plsc

B = 16384
D = 128
NC = 2
NS = 16
NW = NC * NS
BPW = B // NW
CH = 128
NCHUNK = BPW // CH
IDX_ROWS = B // CH


ROWS_PER_IT = 1
NSLOT = 2
GRP = 128
NGRP = CH // GRP


def _sc_body(idx1_hbm, idx2_hbm, table_hbm, out_hbm,
             idx1_v, idx2_v, rows1_v, rows2_v,
             gsem0, gsem1, wsem0, wsem1):
    wid = lax.axis_index("s") * NC + lax.axis_index("c")
    irow = wid * NCHUNK
    pltpu.sync_copy(idx1_hbm.at[pl.ds(irow, NCHUNK)], idx1_v)
    pltpu.sync_copy(idx2_hbm.at[pl.ds(irow, NCHUNK)], idx2_v)

    gsems = (gsem0, gsem1)
    wsems = (wsem0, wsem1)

    def issue_gathers(j):
        slot = j % NSLOT
        c1 = pltpu.async_copy(table_hbm.at[idx1_v.at[j]],
                              rows1_v.at[slot], gsems[slot])
        c2 = pltpu.async_copy(table_hbm.at[idx2_v.at[j]],
                              rows2_v.at[slot], gsems[slot])
        return (c1, c2)

    def accumulate_group(slot, g):
        def add_slice(t, c):
            row = t // (D // 16)
            s = pl.ds((t % (D // 16)) * 16, 16)
            rows1_v[slot, row, s] = (rows1_v[slot, row, s]
                                     + rows2_v[slot, row, s])
            return c
        lax.fori_loop(0, GRP * (D // 16), add_slice, 0, unroll=False)

    def process(j):
        slot = j % NSLOT
        ws = []
        for g in range(NGRP):
            accumulate_group(slot, g)
            ws.append(pltpu.async_copy(
                rows1_v.at[slot, pl.ds(g * GRP, GRP)],
                out_hbm.at[pl.ds((irow + j) * CH + g * GRP, GRP)],
                wsems[slot]))
        return ws

    gathers = [None] * NCHUNK
    writes = [None] * NCHUNK
    gathers[0] = issue_gathers(0)
    for j in range(NCHUNK):
        if j + 1 < NCHUNK:
            if j + 1 >= NSLOT:
                for w in writes[j + 1 - NSLOT]:
                    w.wait()
            gathers[j + 1] = issue_gathers(j + 1)
        c1, c2 = gathers[j]
        c1.wait()
        c2.wait()
        writes[j] = process(j)
    for j in range(max(0, NCHUNK - NSLOT), NCHUNK):
        for w in writes[j]:
            w.wait()


@jax.jit
def _lookup(idx1_2d, idx2_2d, table):
    mesh = plsc.VectorSubcoreMesh(core_axis_name="c", subcore_axis_name="s")
    f = functools.partial(
        pl.kernel, mesh=mesh,
        out_type=jax.ShapeDtypeStruct((B, D), jnp.float32),
        scratch_types=[
            pltpu.VMEM((NCHUNK, CH), jnp.int32),
            pltpu.VMEM((NCHUNK, CH), jnp.int32),
            pltpu.VMEM((NSLOT, CH, D), jnp.float32),
            pltpu.VMEM((NSLOT, CH, D), jnp.float32),
            pltpu.SemaphoreType.DMA,
            pltpu.SemaphoreType.DMA,
            pltpu.SemaphoreType.DMA,
            pltpu.SemaphoreType.DMA,
        ],
    )(_sc_body)
    return f(idx1_2d, idx2_2d, table)


def kernel(idx1, idx2, table):
    idx1_2d = idx1.reshape(IDX_ROWS, CH)
    idx2_2d = idx2.reshape(IDX_ROWS, CH)
    return _lookup(idx1_2d, idx2_2d, table)


# speedup vs baseline: 1.3864x; 1.3864x over previous
import functools

import jax
import jax.numpy as jnp
from jax import lax
from jax.experimental import pallas as pl
from jax.experimental.pallas import tpu as pltpu
from jax.experimental.pallas import tpu_sc as plsc

B = 16384
D = 128
NC = 2
NS = 16
NW = NC * NS
BPW = B // NW
CH = 128
NCHUNK = BPW // CH
IDX_ROWS = B // CH


ROWS_PER_IT = 1
NSLOT = 2
GRP = 128
NGRP = CH // GRP


def _sc_body(idx1_hbm, idx2_hbm, table_hbm, out_hbm,
             idx1_v, idx2_v, rows1_v, rows2_v,
             gsem0, gsem1, wsem0, wsem1):
    wid = lax.axis_index("s") * NC + lax.axis_index("c")
    irow = wid * NCHUNK
    pltpu.sync_copy(idx1_hbm.at[pl.ds(irow, NCHUNK)], idx1_v)
    pltpu.sync_copy(idx2_hbm.at[pl.ds(irow, NCHUNK)], idx2_v)

    gsems = (gsem0, gsem1)
    wsems = (wsem0, wsem1)

    def issue_gathers(j):
        slot = j % NSLOT
        c1 = pltpu.async_copy(table_hbm.at[idx1_v.at[j]],
                              rows1_v.at[slot], gsems[slot])
        c2 = pltpu.async_copy(table_hbm.at[idx2_v.at[j]],
                              rows2_v.at[slot], gsems[slot])
        return (c1, c2)

    def accumulate_group(slot, g):
        def add_rows(i, c):
            for r in range(ROWS_PER_IT):
                row = g * GRP + i * ROWS_PER_IT + r
                for k in range(D // 16):
                    s = pl.ds(k * 16, 16)
                    plsc.addupdate(rows1_v.at[slot, row, s],
                                   rows2_v[slot, row, s])
            return c
        lax.fori_loop(0, GRP // ROWS_PER_IT, add_rows, 0, unroll=False)

    def process(j):
        slot = j % NSLOT
        ws = []
        for g in range(NGRP):
            accumulate_group(slot, g)
            ws.append(pltpu.async_copy(
                rows1_v.at[slot, pl.ds(g * GRP, GRP)],
                out_hbm.at[pl.ds((irow + j) * CH + g * GRP, GRP)],
                wsems[slot]))
        return ws

    gathers = [None] * NCHUNK
    writes = [None] * NCHUNK
    gathers[0] = issue_gathers(0)
    for j in range(NCHUNK):
        if j + 1 < NCHUNK:
            if j + 1 >= NSLOT:
                for w in writes[j + 1 - NSLOT]:
                    w.wait()
            gathers[j + 1] = issue_gathers(j + 1)
        c1, c2 = gathers[j]
        c1.wait()
        c2.wait()
        writes[j] = process(j)
    for j in range(max(0, NCHUNK - NSLOT), NCHUNK):
        for w in writes[j]:
            w.wait()


@jax.jit
def _lookup(idx1_2d, idx2_2d, table):
    mesh = plsc.VectorSubcoreMesh(core_axis_name="c", subcore_axis_name="s")
    f = functools.partial(
        pl.kernel, mesh=mesh,
        out_type=jax.ShapeDtypeStruct((B, D), jnp.float32),
        scratch_types=[
            pltpu.VMEM((NCHUNK, CH), jnp.int32),
            pltpu.VMEM((NCHUNK, CH), jnp.int32),
            pltpu.VMEM((NSLOT, CH, D), jnp.float32),
            pltpu.VMEM((NSLOT, CH, D), jnp.float32),
            pltpu.SemaphoreType.DMA,
            pltpu.SemaphoreType.DMA,
            pltpu.SemaphoreType.DMA,
            pltpu.SemaphoreType.DMA,
        ],
    )(_sc_body)
    return f(idx1_2d, idx2_2d, table)


def kernel(idx1, idx2, table):
    idx1_2d = idx1.reshape(IDX_ROWS, CH)
    idx2_2d = idx2.reshape(IDX_ROWS, CH)
    return _lookup(idx1_2d, idx2_2d, table)
